# baseline (speedup 1.0000x reference)
import functools

import jax
import jax.numpy as jnp
from jax import lax
from jax.experimental import pallas as pl
from jax.experimental.pallas import tpu as pltpu
from jax.experimental.pallas import tpu_sc as plsc

B = 16384
GV = 1000000
LV = 100000
DIM = 32

GTAIL = (GV // 512) * 512
LTAIL0 = (LV // 512) * 512
LTAIL = (LV // 128) * 128
GW = GV // 512
LW = LV // 512


def _transpose_window(slabs, stage, nv=512):
    vgrp = lax.iota(jnp.int32, 16) >> 2
    scol_pat = (lax.iota(jnp.int32, 16) & 3) * 32

    def grp_body(k, carry):
        v0 = 16 * k
        srow = (v0 >> 2) + vgrp
        for s in range(4):
            for f in range(8):
                v = slabs[s][f, pl.ds(v0, 16)]
                plsc.store_scatter(stage, [srow, scol_pat + (8 * s + f)], v)
        return carry

    lax.fori_loop(0, nv // 16, grp_body, 0)


@functools.lru_cache(maxsize=1)
def _build():
    info = plsc.get_sparse_core_info()
    NC, NS = info.num_cores, info.num_subcores
    NW = NC * NS
    mesh = plsc.VectorSubcoreMesh(core_axis_name="c", subcore_axis_name="s")
    cp = pltpu.CompilerParams(needs_layout_passes=False)

    @functools.partial(
        pl.kernel,
        mesh=mesh,
        compiler_params=cp,
        out_type=(
            jax.ShapeDtypeStruct((GV // 4, 128), jnp.float32),
            jax.ShapeDtypeStruct((GV // 4, 128), jnp.float32),
            jax.ShapeDtypeStruct((LV // 4, 128), jnp.float32),
            jax.ShapeDtypeStruct((LV // 4, 128), jnp.float32),
        ),
        scratch_types=[pltpu.VMEM((8, 512), jnp.float32) for _ in range(8)]
        + [
            pltpu.VMEM((128, 132), jnp.float32),
            pltpu.VMEM((128, 132), jnp.float32),
            pltpu.SemaphoreType.DMA,
            pltpu.SemaphoreType.DMA,
        ],
    )
    def retile(WuT, WiT, WaT, WbT, tu, ti, ta, tb, Du, Di, Da, Db,
               s00, s01, s02, s03, s10, s11, s12, s13, st0, st1,
               semA, semB):
        wid = lax.axis_index("s") * NC + lax.axis_index("c")
        slabs = ((s00, s01, s02, s03), (s10, s11, s12, s13))
        stages = (st0, st1)
        sems = (semA, semB)

        def stream_table(WT, D, nwin, per_w):
            w0 = (wid * nwin) // NW
            clamp = nwin - 1

            def issue(j, p):
                win = jnp.minimum(w0 + j, clamp)
                for s in range(4):
                    pltpu.async_copy(
                        WT.at[pl.ds(8 * s, 8), pl.ds(512 * win, 512)],
                        slabs[p][s], sems[p])

            def drain(p):
                for s in range(4):
                    pltpu.make_async_copy(
                        WT.at[pl.ds(0, 8), pl.ds(0, 512)],
                        slabs[p][s], sems[p]).wait()

            issue(0, 0)
            issue(1, 1)

            def pair(jp, carry):
                for p in range(2):
                    j = 2 * jp + p
                    win = jnp.minimum(w0 + j, clamp)
                    drain(p)
                    _transpose_window(slabs[p], stages[p])
                    issue(j + 2, p)
                    pltpu.sync_copy(stages[p].at[:, pl.ds(0, 128)],
                                    D.at[pl.ds(128 * win, 128)])
                return carry

            lax.fori_loop(0, (per_w + 1) // 2, pair, 0)
            drain(0)
            drain(1)

        stream_table(WuT, Du, GW, 62)
        stream_table(WiT, Di, GW, 62)
        stream_table(WaT, Da, LW, 7)
        stream_table(WbT, Db, LW, 7)

        def odd_tile(WT, D):
            for s in range(4):
                pltpu.sync_copy(
                    WT.at[pl.ds(8 * s, 8), pl.ds(LTAIL0, 128)],
                    slabs[0][s].at[:, pl.ds(0, 128)])
            _transpose_window(slabs[0], st0, nv=128)
            pltpu.sync_copy(st0.at[pl.ds(0, 32), pl.ds(0, 128)],
                            D.at[pl.ds(LTAIL0 // 4, 32)])

        @pl.when(wid == 1)
        def _():
            odd_tile(WaT, Da)

        @pl.when(wid == 2)
        def _():
            odd_tile(WbT, Db)

        @pl.when(wid == 3)
        def _():
            pltpu.sync_copy(tu, st0.at[pl.ds(0, 16), pl.ds(0, 128)])
            pltpu.sync_copy(st0.at[pl.ds(0, 16), pl.ds(0, 128)],
                            Du.at[pl.ds(GTAIL // 4, 16)])

        @pl.when(wid == 4)
        def _():
            pltpu.sync_copy(ti, st0.at[pl.ds(0, 16), pl.ds(0, 128)])
            pltpu.sync_copy(st0.at[pl.ds(0, 16), pl.ds(0, 128)],
                            Di.at[pl.ds(GTAIL // 4, 16)])

        @pl.when(wid == 5)
        def _():
            pltpu.sync_copy(ta, st0.at[pl.ds(0, 8), pl.ds(0, 128)])
            pltpu.sync_copy(st0.at[pl.ds(0, 8), pl.ds(0, 128)],
                            Da.at[pl.ds(LTAIL // 4, 8)])

        @pl.when(wid == 6)
        def _():
            pltpu.sync_copy(tb, st0.at[pl.ds(0, 8), pl.ds(0, 128)])
            pltpu.sync_copy(st0.at[pl.ds(0, 8), pl.ds(0, 128)],
                            Db.at[pl.ds(LTAIL // 4, 8)])

    NB = 64

    @functools.partial(
        pl.kernel,
        mesh=mesh,
        compiler_params=cp,
        out_type=(
            jax.ShapeDtypeStruct((2 * DIM, B), jnp.float32),
            jax.ShapeDtypeStruct((2 * DIM, B), jnp.float32),
        ),
        scratch_types=[pltpu.VMEM((512,), jnp.int32) for _ in range(4)]
        + [pltpu.VMEM((NB,), jnp.int32) for _ in range(4)]
        + [pltpu.VMEM((NB, 128), jnp.float32) for _ in range(4)]
        + [pltpu.VMEM((NB, 132), jnp.float32) for _ in range(4)]
        + [
            pltpu.VMEM((2 * DIM, 128), jnp.float32),
            pltpu.VMEM((2 * DIM, 128), jnp.float32),
            pltpu.SemaphoreType.DMA,
        ],
    )
    def gather(Du, Di, Da, Db, uid, iid, ca, cb, gT, lT,
               xu, xi, xa, xb, qu, qi, qa, qb,
               ru, ri, ra, rb, pu, pi, pa, pb, ag, al, sem):
        wid = lax.axis_index("s") * NC + lax.axis_index("c")
        base = wid * 512
        pltpu.sync_copy(uid.at[pl.ds(base, 512)], xu)
        pltpu.sync_copy(iid.at[pl.ds(base, 512)], xi)
        pltpu.sync_copy(ca.at[pl.ds(base, 512)], xa)
        pltpu.sync_copy(cb.at[pl.ds(base, 512)], xb)
        lanes = lax.iota(jnp.int32, 16)

        for c in range(512 // NB):
            for x, q in ((xu, qu), (xi, qi), (xa, qa), (xb, qb)):
                for j in range(NB // 16):
                    q[pl.ds(16 * j, 16)] = x[pl.ds(NB * c + 16 * j, 16)] >> 2
            cps = [pltpu.async_copy(D.at[q], r, sem)
                   for D, q, r in ((Du, qu, ru), (Di, qi, ri),
                                   (Da, qa, ra), (Db, qb, rb))]
            for h in cps:
                h.wait()
            def rp_body(r2, carry):
                for r, p in ((ru, pu), (ri, pi), (ra, pa), (rb, pb)):
                    for j in range(8):
                        p[r2, pl.ds(16 * j, 16)] = r[r2, pl.ds(16 * j, 16)]
                return carry

            lax.fori_loop(0, NB, rp_body, 0)
            for x, q in ((xu, qu), (xi, qi), (xa, qa), (xb, qb)):
                for j in range(NB // 16):
                    q[pl.ds(16 * j, 16)] = (
                        x[pl.ds(NB * c + 16 * j, 16)] & 3) * 32

            half = (c % 2) * NB

            def f_body(f, carry):
                for j in range(NB // 16):
                    rows = lanes + 16 * j
                    d = pl.ds(half + 16 * j, 16)
                    ag[f, d] = plsc.load_gather(
                        pu, [rows, qu[pl.ds(16 * j, 16)] + f])
                    ag[DIM + f, d] = plsc.load_gather(
                        pi, [rows, qi[pl.ds(16 * j, 16)] + f])
                    al[f, d] = plsc.load_gather(
                        pa, [rows, qa[pl.ds(16 * j, 16)] + f])
                    al[DIM + f, d] = plsc.load_gather(
                        pb, [rows, qb[pl.ds(16 * j, 16)] + f])
                return carry

            lax.fori_loop(0, DIM, f_body, 0)
            if c % 2 == 1:
                w0 = base + NB * (c - 1)
                pltpu.sync_copy(ag, gT.at[pl.ds(0, 2 * DIM),
                                          pl.ds(w0, 128)])
                pltpu.sync_copy(al, lT.at[pl.ds(0, 2 * DIM),
                                          pl.ds(w0, 128)])

    return retile, gather


def kernel(W_user, W_item, W_cat_a, W_cat_b, user_id, item_id, cat_a, cat_b):
    retile, gather = _build()
    tu = W_user[GTAIL:].reshape(16, 128)
    ti = W_item[GTAIL:].reshape(16, 128)
    ta = W_cat_a[LTAIL:].reshape(8, 128)
    tb = W_cat_b[LTAIL:].reshape(8, 128)
    Du, Di, Da, Db = retile(W_user.T, W_item.T, W_cat_a.T, W_cat_b.T,
                            tu, ti, ta, tb)
    gT, lT = gather(Du, Di, Da, Db,
                    user_id.astype(jnp.int32), item_id.astype(jnp.int32),
                    cat_a.astype(jnp.int32), cat_b.astype(jnp.int32))
    return gT.T, lT.T


# speedup vs baseline: 1.7400x; 1.7400x over previous
import functools

import jax
import jax.numpy as jnp
from jax import lax
from jax.experimental import pallas as pl
from jax.experimental.pallas import tpu as pltpu
from jax.experimental.pallas import tpu_sc as plsc

B = 16384
GLOBAL_VOCAB = 1000000
LOCAL_VOCAB = 100000
DIM = 32


@functools.lru_cache(maxsize=1)
def _build():
    info = plsc.get_sparse_core_info()
    NC, NS = info.num_cores, info.num_subcores
    NW = NC * NS
    bpw = B // NW
    mesh = plsc.VectorSubcoreMesh(core_axis_name="c", subcore_axis_name="s")

    @functools.partial(
        pl.kernel,
        mesh=mesh,
        compiler_params=pltpu.CompilerParams(use_tc_tiling_on_sc=False,
                                             needs_layout_passes=False),
        out_type=(
            jax.ShapeDtypeStruct((4 * B, 16), jnp.float32),
            jax.ShapeDtypeStruct((4 * B, 16), jnp.float32),
        ),
        scratch_types=[
            pltpu.VMEM((bpw,), jnp.int32),
            pltpu.VMEM((bpw,), jnp.int32),
            pltpu.VMEM((bpw,), jnp.int32),
            pltpu.VMEM((bpw,), jnp.int32),
            pltpu.VMEM((2 * bpw,), jnp.int32),
            pltpu.VMEM((2 * bpw,), jnp.int32),
            pltpu.VMEM((2 * bpw,), jnp.int32),
            pltpu.VMEM((2 * bpw,), jnp.int32),
            pltpu.VMEM((2 * bpw,), jnp.int32),
            pltpu.VMEM((2 * bpw,), jnp.int32),
            pltpu.VMEM((2 * bpw, 16), jnp.float32),
            pltpu.VMEM((2 * bpw, 16), jnp.float32),
            pltpu.VMEM((2 * bpw, 16), jnp.float32),
            pltpu.VMEM((2 * bpw, 16), jnp.float32),
            pltpu.SemaphoreType.DMA,
            pltpu.SemaphoreType.DMA,
        ],
    )
    def k(Wu, Wi, Wa, Wb, uid, iid, ca, cb, ou_hbm, oi_hbm, g_out, l_out,
          idx_u, idx_i, idx_a, idx_b,
          x2_u, x2_i, x2_a, x2_b, oidx_u, oidx_i,
          r_u, r_i, r_a, r_b, sem, sem2):
        wid = lax.axis_index("s") * NC + lax.axis_index("c")
        base = wid * bpw
        pltpu.sync_copy(uid.at[pl.ds(base, bpw)], idx_u)
        pltpu.sync_copy(iid.at[pl.ds(base, bpw)], idx_i)
        pltpu.sync_copy(ca.at[pl.ds(base, bpw)], idx_a)
        pltpu.sync_copy(cb.at[pl.ds(base, bpw)], idx_b)
        pltpu.sync_copy(ou_hbm.at[pl.ds(2 * base, 2 * bpw)], oidx_u)
        pltpu.sync_copy(oi_hbm.at[pl.ds(2 * base, 2 * bpw)], oidx_i)

        lane = lax.iota(jnp.int32, 16)

        def expand(c, _):
            pos = 32 * c + 2 * lane
            for src, dst in ((idx_u, x2_u), (idx_i, x2_i),
                             (idx_a, x2_a), (idx_b, x2_b)):
                v = 2 * src[pl.ds(c * 16, 16)]
                plsc.store_scatter(dst, [pos], v)
                plsc.store_scatter(dst, [pos + 1], v + 1)
            return _

        lax.fori_loop(0, bpw // 16, expand, 0)

        du = pltpu.async_copy(Wu.at[x2_u], r_u, sem)
        di = pltpu.async_copy(Wi.at[x2_i], r_i, sem)
        da = pltpu.async_copy(Wa.at[x2_a], r_a, sem)
        db = pltpu.async_copy(Wb.at[x2_b], r_b, sem)
        du.wait()
        su = pltpu.async_copy(r_u, g_out.at[oidx_u], sem2)
        di.wait()
        si = pltpu.async_copy(r_i, g_out.at[oidx_i], sem2)
        da.wait()
        sa = pltpu.async_copy(r_a, l_out.at[oidx_u], sem2)
        db.wait()
        sb = pltpu.async_copy(r_b, l_out.at[oidx_i], sem2)
        su.wait()
        si.wait()
        sa.wait()
        sb.wait()

    return k


def kernel(W_user, W_item, W_cat_a, W_cat_b, user_id, item_id, cat_a, cat_b):
    k = _build()
    b4 = 4 * jnp.arange(B, dtype=jnp.int32)
    ou = jnp.stack([b4, b4 + 1], axis=1).reshape(-1)
    oi = ou + 2
    g4, l4 = k(W_user.reshape(-1, 16), W_item.reshape(-1, 16),
               W_cat_a.reshape(-1, 16), W_cat_b.reshape(-1, 16),
               user_id.astype(jnp.int32), item_id.astype(jnp.int32),
               cat_a.astype(jnp.int32), cat_b.astype(jnp.int32),
               ou, oi)
    return (g4.reshape(B, 2 * DIM), l4.reshape(B, 2 * DIM))
